# baseline (speedup 1.0000x reference)
import functools

import jax
import jax.numpy as jnp
from jax import lax
from jax.experimental import pallas as pl
from jax.experimental.pallas import tpu as pltpu
from jax.experimental.pallas import tpu_sc as plsc

N = 10000
E = 320000
D = 128

NC = 2
NS = 16
NW = NC * NS
CHUNK = 128
NCHT = E // CHUNK
NCH_BASE = NCHT // NW
NCH_REM = NCHT - NCH_BASE * NW

AP = 10240
RPT = AP // NS
WCHUNK = CHUNK
NWCHUNK = RPT // WCHUNK

CNT_LEN = 3 * AP
ZSH = CNT_LEN // NS

HC = CHUNK // 2
BLKN = 1024
YBLK = 1000


def _ybuild_body(x_ref, w_ref, y_ref):
    x = x_ref[...]
    for t in range(3):
        y_ref[t] = lax.dot_general(
            x, w_ref[t], (((1,), (1,)), ((), ())),
            preferred_element_type=jnp.float32)


def _edge_body(y_hbm, ei_hbm, ef_hbm, a_out, c_out,
               ei_v, ef_v, gidx_v, cidx_v, ones_v,
               rows_v, zcnt_v, a_sh, c_sh,
               semi, semg, sems):
    c = lax.axis_index("c")
    s = lax.axis_index("s")
    wid = c * NS + s
    nch = NCH_BASE + jnp.where(wid < NCH_REM, 1, 0)

    zf = jnp.zeros((16,), jnp.float32)
    for j in range(CHUNK // 16):
        ones_v[pl.ds(j * 16, 16)] = jnp.ones((16,), jnp.float32)

    def _zc(i, carry):
        zcnt_v[pl.ds(i * 16, 16)] = zf
        return carry
    lax.fori_loop(0, ZSH // 16, _zc, 0)
    pltpu.sync_copy(zcnt_v, c_sh.at[pl.ds(s * ZSH, ZSH)])

    def _calc_idx(buf, slot):
        for j in range(CHUNK // 16):
            sl = pl.ds(j * 16, 16)
            sv = ei_v[slot, 0, sl]
            dv = ei_v[slot, 1, sl]
            ev = ef_v[slot, sl]
            gidx_v[buf, sl] = ev * AP + sv
            cidx_v[slot, sl] = ev * AP + dv

    base0 = wid * CHUNK
    pltpu.sync_copy(ei_hbm.at[:, pl.ds(base0, CHUNK)], ei_v.at[0])
    pltpu.sync_copy(ef_hbm.at[pl.ds(base0, CHUNK)], ef_v.at[0])
    _calc_idx(0, 0)
    pltpu.async_copy(y_hbm.at[gidx_v.at[0, pl.ds(0, HC)]],
                     rows_v.at[0, pl.ds(0, HC)], semg)
    pltpu.async_copy(y_hbm.at[gidx_v.at[0, pl.ds(HC, HC)]],
                     rows_v.at[0, pl.ds(HC, HC)], semg)
    base1 = (NW + wid) * CHUNK
    pltpu.async_copy(ei_hbm.at[:, pl.ds(base1, CHUNK)], ei_v.at[1], semi)
    pltpu.async_copy(ef_hbm.at[pl.ds(base1, CHUNK)], ef_v.at[1], semi)

    def _zr(r, carry):
        for j in range(D // 16):
            rows_v[1, r, pl.ds(j * 16, 16)] = zf
        return carry
    lax.fori_loop(0, CHUNK, _zr, 0)

    def _za(k, carry):
        pltpu.async_copy(rows_v.at[1], a_sh.at[pl.ds(s * RPT + k * CHUNK,
                                                     CHUNK)], sems)
        return carry
    lax.fori_loop(0, RPT // CHUNK, _za, 0)
    for k in range(RPT // CHUNK):
        pltpu.make_async_copy(rows_v.at[1], a_sh.at[pl.ds(0, CHUNK)],
                              sems).wait()

    plsc.subcore_barrier()

    def _chunk(g, carry):
        b = lax.rem(g, 2)
        nb = 1 - b
        t = lax.rem(g, 3)
        tn = lax.rem(g + 1, 3)
        tp = lax.rem(g + 2, 3)
        do_g1 = (g + 1) < nch
        do_l2 = (g + 2) < nch

        @pl.when(g > 0)
        def _drain_prev():
            pltpu.make_async_copy(rows_v.at[nb], a_sh.at[ei_v.at[tp, 1]],
                                  sems).wait()
            pltpu.make_async_copy(ones_v, c_sh.at[cidx_v.at[tp]], sems).wait()

        @pl.when(do_l2)
        def _pf_loads():
            base = ((g + 2) * NW + wid) * CHUNK
            pltpu.async_copy(ei_hbm.at[:, pl.ds(base, CHUNK)], ei_v.at[tp],
                             semi)
            pltpu.async_copy(ef_hbm.at[pl.ds(base, CHUNK)], ef_v.at[tp], semi)

        @pl.when(do_g1)
        def _pf_gather():
            pltpu.make_async_copy(ei_hbm.at[:, pl.ds(0, CHUNK)], ei_v.at[tn],
                                  semi).wait()
            pltpu.make_async_copy(ef_hbm.at[pl.ds(0, CHUNK)], ef_v.at[tn],
                                  semi).wait()
            _calc_idx(nb, tn)
            pltpu.async_copy(y_hbm.at[gidx_v.at[nb, pl.ds(0, HC)]],
                             rows_v.at[nb, pl.ds(0, HC)], semg)
            pltpu.async_copy(y_hbm.at[gidx_v.at[nb, pl.ds(HC, HC)]],
                             rows_v.at[nb, pl.ds(HC, HC)], semg)

        pltpu.make_async_copy(y_hbm.at[pl.ds(0, HC)],
                              rows_v.at[b, pl.ds(0, HC)], semg).wait()
        pltpu.make_async_copy(y_hbm.at[pl.ds(0, HC)],
                              rows_v.at[b, pl.ds(HC, HC)], semg).wait()
        pltpu.async_copy(rows_v.at[b], a_sh.at[ei_v.at[t, 1]], sems, add=True)
        pltpu.async_copy(ones_v, c_sh.at[cidx_v.at[t]], sems, add=True)
        return carry
    lax.fori_loop(0, nch, _chunk, 0)

    lastb = lax.rem(nch - 1, 2)
    lastt = lax.rem(nch - 1, 3)
    pltpu.make_async_copy(rows_v.at[lastb], a_sh.at[ei_v.at[lastt, 1]],
                          sems).wait()
    pltpu.make_async_copy(ones_v, c_sh.at[cidx_v.at[lastt]], sems).wait()

    plsc.subcore_barrier()

    for k in range(NWCHUNK):
        r0 = s * RPT + k * WCHUNK
        pltpu.sync_copy(a_sh.at[pl.ds(r0, WCHUNK)], rows_v.at[k % 2])
        pltpu.sync_copy(rows_v.at[k % 2], a_out.at[c, pl.ds(r0, WCHUNK)])
    pltpu.sync_copy(c_sh.at[pl.ds(s * ZSH, ZSH)], zcnt_v)
    pltpu.sync_copy(zcnt_v, c_out.at[c, pl.ds(s * ZSH, ZSH)])


def _combine_body(a_ref, c_ref, x_ref, w_ref, wh_ref, b3_ref, bh_ref, o_ref):
    a = a_ref[0] + a_ref[1]
    cnt = c_ref[0] + c_ref[1]
    ct = jnp.transpose(cnt, (1, 0))
    x = x_ref[...]
    corr = jnp.zeros_like(a)
    for t in range(3):
        yt = lax.dot_general(x, w_ref[t], (((1,), (1,)), ((), ())),
                             preferred_element_type=jnp.float32)
        corr = corr + ct[:, t:t + 1] * yt
    bterm = lax.dot_general(ct, b3_ref[...], (((1,), (0,)), ((), ())),
                            preferred_element_type=jnp.float32)
    yh = lax.dot_general(x, wh_ref[...], (((1,), (1,)), ((), ())),
                         preferred_element_type=jnp.float32)
    o_ref[...] = a - corr + bterm + yh + bh_ref[...]


_sc_mesh = plsc.VectorSubcoreMesh(
    core_axis_name="c", subcore_axis_name="s", num_cores=NC, num_subcores=NS)

_edge_pass = functools.partial(
    pl.kernel,
    out_type=[jax.ShapeDtypeStruct((NC, AP, D), jnp.float32),
              jax.ShapeDtypeStruct((NC, CNT_LEN), jnp.float32)],
    mesh=_sc_mesh,
    scratch_types=[
        pltpu.VMEM((3, 2, CHUNK), jnp.int32),
        pltpu.VMEM((3, CHUNK), jnp.int32),
        pltpu.VMEM((2, CHUNK), jnp.int32),
        pltpu.VMEM((3, CHUNK), jnp.int32),
        pltpu.VMEM((CHUNK,), jnp.float32),
        pltpu.VMEM((2, CHUNK, D), jnp.float32),
        pltpu.VMEM((ZSH,), jnp.float32),
        pltpu.VMEM_SHARED((AP, D), jnp.float32),
        pltpu.VMEM_SHARED((CNT_LEN,), jnp.float32),
        pltpu.SemaphoreType.DMA,
        pltpu.SemaphoreType.DMA,
        pltpu.SemaphoreType.DMA,
    ],
)(_edge_body)


def kernel(n_in_feats, edge_index, e_feats, W0, b0, W1, b1, W2, b2, Wh, bh):
    x = n_in_feats
    W = jnp.stack([W0, W1, W2])
    B3 = jnp.stack([b0, b1, b2])

    y3 = pl.pallas_call(
        _ybuild_body,
        grid=(N // YBLK,),
        in_specs=[pl.BlockSpec((YBLK, D), lambda i: (i, 0)),
                  pl.BlockSpec((3, D, D), lambda i: (0, 0, 0))],
        out_specs=pl.BlockSpec((3, YBLK, D), lambda i: (0, i, 0)),
        out_shape=jax.ShapeDtypeStruct((3, AP, D), jnp.float32),
    )(x, W)

    a_part, cnt_flat = _edge_pass(y3.reshape(3 * AP, D), edge_index, e_feats)
    cnt3 = cnt_flat.reshape(NC, 3, AP)

    out = pl.pallas_call(
        _combine_body,
        grid=(AP // BLKN,),
        in_specs=[pl.BlockSpec((NC, BLKN, D), lambda i: (0, i, 0)),
                  pl.BlockSpec((NC, 3, BLKN), lambda i: (0, 0, i)),
                  pl.BlockSpec((BLKN, D), lambda i: (i, 0)),
                  pl.BlockSpec((3, D, D), lambda i: (0, 0, 0)),
                  pl.BlockSpec((D, D), lambda i: (0, 0)),
                  pl.BlockSpec((3, D), lambda i: (0, 0)),
                  pl.BlockSpec((1, D), lambda i: (0, 0))],
        out_specs=pl.BlockSpec((BLKN, D), lambda i: (i, 0)),
        out_shape=jax.ShapeDtypeStruct((N, D), jnp.float32),
    )(a_part, cnt3, x, W, Wh, B3, bh.reshape(1, D))
    return out


# speedup vs baseline: 1.0106x; 1.0106x over previous
import functools

import jax
import jax.numpy as jnp
from jax import lax
from jax.experimental import pallas as pl
from jax.experimental.pallas import tpu as pltpu
from jax.experimental.pallas import tpu_sc as plsc

N = 10000
E = 320000
D = 128

NC = 2
NS = 16
NW = NC * NS
CHUNK = 128
NCHT = E // CHUNK
NCH_BASE = NCHT // NW
NCH_REM = NCHT - NCH_BASE * NW

AP = 10240
RPT = AP // NS
WCHUNK = CHUNK
NWCHUNK = RPT // WCHUNK

CNT_LEN = 3 * AP
ZSH = CNT_LEN // NS

BLKN = 1024
YBLK = 1000


def _ybuild_body(x_ref, w_ref, y_ref):
    x = x_ref[...]
    for t in range(3):
        y_ref[t] = lax.dot_general(
            x, w_ref[t], (((1,), (1,)), ((), ())),
            preferred_element_type=jnp.float32)


def _edge_body(y_hbm, ei_hbm, ef_hbm, a_out, c_out,
               ei_v, ef_v, gidx_v, cidx_v, ones_v,
               rows_v, zcnt_v, a_sh, c_sh,
               semi, semg, sems):
    c = lax.axis_index("c")
    s = lax.axis_index("s")
    wid = c * NS + s
    nch = NCH_BASE + jnp.where(wid < NCH_REM, 1, 0)

    zf = jnp.zeros((16,), jnp.float32)
    for j in range(CHUNK // 16):
        ones_v[pl.ds(j * 16, 16)] = jnp.ones((16,), jnp.float32)

    def _zc(i, carry):
        zcnt_v[pl.ds(i * 16, 16)] = zf
        return carry
    lax.fori_loop(0, ZSH // 16, _zc, 0)
    pltpu.sync_copy(zcnt_v, c_sh.at[pl.ds(s * ZSH, ZSH)])

    def _calc_idx(buf, slot):
        for j in range(CHUNK // 16):
            sl = pl.ds(j * 16, 16)
            sv = ei_v[slot, 0, sl]
            dv = ei_v[slot, 1, sl]
            ev = ef_v[slot, sl]
            gidx_v[buf, sl] = ev * AP + sv
            cidx_v[slot, sl] = ev * AP + dv

    base0 = wid * CHUNK
    pltpu.sync_copy(ei_hbm.at[:, pl.ds(base0, CHUNK)], ei_v.at[0])
    pltpu.sync_copy(ef_hbm.at[pl.ds(base0, CHUNK)], ef_v.at[0])
    _calc_idx(0, 0)
    pltpu.async_copy(y_hbm.at[gidx_v.at[0]], rows_v.at[0], semg)
    base1 = (NW + wid) * CHUNK
    pltpu.async_copy(ei_hbm.at[:, pl.ds(base1, CHUNK)], ei_v.at[1], semi)
    pltpu.async_copy(ef_hbm.at[pl.ds(base1, CHUNK)], ef_v.at[1], semi)

    def _zr(r, carry):
        for j in range(D // 16):
            rows_v[1, r, pl.ds(j * 16, 16)] = zf
        return carry
    lax.fori_loop(0, CHUNK, _zr, 0)

    def _za(k, carry):
        pltpu.async_copy(rows_v.at[1], a_sh.at[pl.ds(s * RPT + k * CHUNK,
                                                     CHUNK)], sems)
        return carry
    lax.fori_loop(0, RPT // CHUNK, _za, 0)
    for k in range(RPT // CHUNK):
        pltpu.make_async_copy(rows_v.at[1], a_sh.at[pl.ds(0, CHUNK)],
                              sems).wait()

    plsc.subcore_barrier()

    def _chunk(g, carry):
        b = lax.rem(g, 2)
        nb = 1 - b
        t = lax.rem(g, 3)
        tn = lax.rem(g + 1, 3)
        tp = lax.rem(g + 2, 3)
        do_g1 = (g + 1) < nch
        do_l2 = (g + 2) < nch

        @pl.when(g > 0)
        def _drain_prev():
            pltpu.make_async_copy(rows_v.at[nb], a_sh.at[ei_v.at[tp, 1]],
                                  sems).wait()
            pltpu.make_async_copy(ones_v, c_sh.at[cidx_v.at[tp]], sems).wait()

        @pl.when(do_l2)
        def _pf_loads():
            base = ((g + 2) * NW + wid) * CHUNK
            pltpu.async_copy(ei_hbm.at[:, pl.ds(base, CHUNK)], ei_v.at[tp],
                             semi)
            pltpu.async_copy(ef_hbm.at[pl.ds(base, CHUNK)], ef_v.at[tp], semi)

        @pl.when(do_g1)
        def _pf_gather():
            pltpu.make_async_copy(ei_hbm.at[:, pl.ds(0, CHUNK)], ei_v.at[tn],
                                  semi).wait()
            pltpu.make_async_copy(ef_hbm.at[pl.ds(0, CHUNK)], ef_v.at[tn],
                                  semi).wait()
            _calc_idx(nb, tn)
            pltpu.async_copy(y_hbm.at[gidx_v.at[nb]], rows_v.at[nb], semg)

        pltpu.make_async_copy(y_hbm.at[pl.ds(0, CHUNK)], rows_v.at[b], semg).wait()
        pltpu.async_copy(rows_v.at[b], a_sh.at[ei_v.at[t, 1]], sems, add=True)
        pltpu.async_copy(ones_v, c_sh.at[cidx_v.at[t]], sems, add=True)
        return carry
    lax.fori_loop(0, nch, _chunk, 0)

    lastb = lax.rem(nch - 1, 2)
    lastt = lax.rem(nch - 1, 3)
    pltpu.make_async_copy(rows_v.at[lastb], a_sh.at[ei_v.at[lastt, 1]],
                          sems).wait()
    pltpu.make_async_copy(ones_v, c_sh.at[cidx_v.at[lastt]], sems).wait()

    plsc.subcore_barrier()

    for k in range(NWCHUNK):
        r0 = s * RPT + k * WCHUNK
        if k >= 2:
            pltpu.make_async_copy(rows_v.at[k % 2],
                                  a_out.at[c, pl.ds(0, WCHUNK)], sems).wait()
        pltpu.sync_copy(a_sh.at[pl.ds(r0, WCHUNK)], rows_v.at[k % 2])
        pltpu.async_copy(rows_v.at[k % 2], a_out.at[c, pl.ds(r0, WCHUNK)],
                         sems)
    pltpu.sync_copy(c_sh.at[pl.ds(s * ZSH, ZSH)], zcnt_v)
    pltpu.async_copy(zcnt_v, c_out.at[c, pl.ds(s * ZSH, ZSH)], sems)
    for k in range(NWCHUNK - 2, NWCHUNK):
        pltpu.make_async_copy(rows_v.at[k % 2],
                              a_out.at[c, pl.ds(0, WCHUNK)], sems).wait()
    pltpu.make_async_copy(zcnt_v, c_out.at[c, pl.ds(0, ZSH)], sems).wait()


def _combine_body(a_ref, c_ref, x_ref, w_ref, wh_ref, b3_ref, bh_ref, o_ref):
    a = a_ref[0] + a_ref[1]
    cnt = c_ref[0] + c_ref[1]
    ct = jnp.transpose(cnt, (1, 0))
    x = x_ref[...]
    corr = jnp.zeros_like(a)
    for t in range(3):
        yt = lax.dot_general(x, w_ref[t], (((1,), (1,)), ((), ())),
                             preferred_element_type=jnp.float32)
        corr = corr + ct[:, t:t + 1] * yt
    bterm = lax.dot_general(ct, b3_ref[...], (((1,), (0,)), ((), ())),
                            preferred_element_type=jnp.float32)
    yh = lax.dot_general(x, wh_ref[...], (((1,), (1,)), ((), ())),
                         preferred_element_type=jnp.float32)
    o_ref[...] = a - corr + bterm + yh + bh_ref[...]


_sc_mesh = plsc.VectorSubcoreMesh(
    core_axis_name="c", subcore_axis_name="s", num_cores=NC, num_subcores=NS)

_edge_pass = functools.partial(
    pl.kernel,
    out_type=[jax.ShapeDtypeStruct((NC, AP, D), jnp.float32),
              jax.ShapeDtypeStruct((NC, CNT_LEN), jnp.float32)],
    mesh=_sc_mesh,
    scratch_types=[
        pltpu.VMEM((3, 2, CHUNK), jnp.int32),
        pltpu.VMEM((3, CHUNK), jnp.int32),
        pltpu.VMEM((2, CHUNK), jnp.int32),
        pltpu.VMEM((3, CHUNK), jnp.int32),
        pltpu.VMEM((CHUNK,), jnp.float32),
        pltpu.VMEM((2, CHUNK, D), jnp.float32),
        pltpu.VMEM((ZSH,), jnp.float32),
        pltpu.VMEM_SHARED((AP, D), jnp.float32),
        pltpu.VMEM_SHARED((CNT_LEN,), jnp.float32),
        pltpu.SemaphoreType.DMA,
        pltpu.SemaphoreType.DMA,
        pltpu.SemaphoreType.DMA,
    ],
)(_edge_body)


def kernel(n_in_feats, edge_index, e_feats, W0, b0, W1, b1, W2, b2, Wh, bh):
    x = n_in_feats
    W = jnp.stack([W0, W1, W2])
    B3 = jnp.stack([b0, b1, b2])

    y3 = pl.pallas_call(
        _ybuild_body,
        grid=(N // YBLK,),
        in_specs=[pl.BlockSpec((YBLK, D), lambda i: (i, 0)),
                  pl.BlockSpec((3, D, D), lambda i: (0, 0, 0))],
        out_specs=pl.BlockSpec((3, YBLK, D), lambda i: (0, i, 0)),
        out_shape=jax.ShapeDtypeStruct((3, AP, D), jnp.float32),
    )(x, W)

    a_part, cnt_flat = _edge_pass(y3.reshape(3 * AP, D), edge_index, e_feats)
    cnt3 = cnt_flat.reshape(NC, 3, AP)

    out = pl.pallas_call(
        _combine_body,
        grid=(AP // BLKN,),
        in_specs=[pl.BlockSpec((NC, BLKN, D), lambda i: (0, i, 0)),
                  pl.BlockSpec((NC, 3, BLKN), lambda i: (0, 0, i)),
                  pl.BlockSpec((BLKN, D), lambda i: (i, 0)),
                  pl.BlockSpec((3, D, D), lambda i: (0, 0, 0)),
                  pl.BlockSpec((D, D), lambda i: (0, 0)),
                  pl.BlockSpec((3, D), lambda i: (0, 0)),
                  pl.BlockSpec((1, D), lambda i: (0, 0))],
        out_specs=pl.BlockSpec((BLKN, D), lambda i: (i, 0)),
        out_shape=jax.ShapeDtypeStruct((N, D), jnp.float32),
    )(a_part, cnt3, x, W, Wh, B3, bh.reshape(1, D))
    return out


# speedup vs baseline: 1.0269x; 1.0162x over previous
import functools

import jax
import jax.numpy as jnp
from jax import lax
from jax.experimental import pallas as pl
from jax.experimental.pallas import tpu as pltpu
from jax.experimental.pallas import tpu_sc as plsc

N = 10000
E = 320000
D = 128

NC = 2
NS = 16
NW = NC * NS
CHUNK = 128
NCHT = E // CHUNK
NCH_BASE = NCHT // NW
NCH_REM = NCHT - NCH_BASE * NW

AP = 10240
RPT = AP // NS
WCHUNK = CHUNK
NWCHUNK = RPT // WCHUNK

CNT_LEN = 3 * AP
ZSH = CNT_LEN // NS

BLKN = 1024
YBLK = 1000


def _ybuild_body(x_ref, w0_ref, w1_ref, w2_ref, y_ref):
    x = x_ref[...]
    for t, wr in enumerate((w0_ref, w1_ref, w2_ref)):
        y_ref[t] = lax.dot_general(
            x, wr[...], (((1,), (1,)), ((), ())),
            preferred_element_type=jnp.float32)


def _edge_body(y_hbm, ei_hbm, ef_hbm, a_out, c_out,
               ei_v, ef_v, gidx_v, cidx_v, ones_v,
               rows_v, zcnt_v, a_sh, c_sh,
               semi, semg, sems):
    c = lax.axis_index("c")
    s = lax.axis_index("s")
    wid = c * NS + s
    nch = NCH_BASE + jnp.where(wid < NCH_REM, 1, 0)

    zf = jnp.zeros((16,), jnp.float32)
    for j in range(CHUNK // 16):
        ones_v[pl.ds(j * 16, 16)] = jnp.ones((16,), jnp.float32)

    def _zc(i, carry):
        zcnt_v[pl.ds(i * 16, 16)] = zf
        return carry
    lax.fori_loop(0, ZSH // 16, _zc, 0)
    pltpu.sync_copy(zcnt_v, c_sh.at[pl.ds(s * ZSH, ZSH)])

    def _calc_idx(buf, slot):
        for j in range(CHUNK // 16):
            sl = pl.ds(j * 16, 16)
            sv = ei_v[slot, 0, sl]
            dv = ei_v[slot, 1, sl]
            ev = ef_v[slot, sl]
            gidx_v[buf, sl] = ev * AP + sv
            cidx_v[slot, sl] = ev * AP + dv

    base0 = wid * CHUNK
    pltpu.sync_copy(ei_hbm.at[:, pl.ds(base0, CHUNK)], ei_v.at[0])
    pltpu.sync_copy(ef_hbm.at[pl.ds(base0, CHUNK)], ef_v.at[0])
    _calc_idx(0, 0)
    pltpu.async_copy(y_hbm.at[gidx_v.at[0]], rows_v.at[0], semg)
    base1 = (NW + wid) * CHUNK
    pltpu.async_copy(ei_hbm.at[:, pl.ds(base1, CHUNK)], ei_v.at[1], semi)
    pltpu.async_copy(ef_hbm.at[pl.ds(base1, CHUNK)], ef_v.at[1], semi)

    def _zr(r, carry):
        for j in range(D // 16):
            rows_v[1, r, pl.ds(j * 16, 16)] = zf
        return carry
    lax.fori_loop(0, CHUNK, _zr, 0)

    def _za(k, carry):
        pltpu.async_copy(rows_v.at[1], a_sh.at[pl.ds(s * RPT + k * CHUNK,
                                                     CHUNK)], sems)
        return carry
    lax.fori_loop(0, RPT // CHUNK, _za, 0)
    for k in range(RPT // CHUNK):
        pltpu.make_async_copy(rows_v.at[1], a_sh.at[pl.ds(0, CHUNK)],
                              sems).wait()

    plsc.subcore_barrier()

    def _chunk(g, carry):
        b = lax.rem(g, 2)
        nb = 1 - b
        t = lax.rem(g, 3)
        tn = lax.rem(g + 1, 3)
        tp = lax.rem(g + 2, 3)
        do_g1 = (g + 1) < nch
        do_l2 = (g + 2) < nch

        @pl.when(g > 0)
        def _drain_prev():
            pltpu.make_async_copy(rows_v.at[nb], a_sh.at[ei_v.at[tp, 1]],
                                  sems).wait()
            pltpu.make_async_copy(ones_v, c_sh.at[cidx_v.at[tp]], sems).wait()

        @pl.when(do_l2)
        def _pf_loads():
            base = ((g + 2) * NW + wid) * CHUNK
            pltpu.async_copy(ei_hbm.at[:, pl.ds(base, CHUNK)], ei_v.at[tp],
                             semi)
            pltpu.async_copy(ef_hbm.at[pl.ds(base, CHUNK)], ef_v.at[tp], semi)

        @pl.when(do_g1)
        def _pf_gather():
            pltpu.make_async_copy(ei_hbm.at[:, pl.ds(0, CHUNK)], ei_v.at[tn],
                                  semi).wait()
            pltpu.make_async_copy(ef_hbm.at[pl.ds(0, CHUNK)], ef_v.at[tn],
                                  semi).wait()
            _calc_idx(nb, tn)
            pltpu.async_copy(y_hbm.at[gidx_v.at[nb]], rows_v.at[nb], semg)

        pltpu.make_async_copy(y_hbm.at[pl.ds(0, CHUNK)], rows_v.at[b], semg).wait()
        pltpu.async_copy(rows_v.at[b], a_sh.at[ei_v.at[t, 1]], sems, add=True)
        pltpu.async_copy(ones_v, c_sh.at[cidx_v.at[t]], sems, add=True)
        return carry
    lax.fori_loop(0, nch, _chunk, 0)

    lastb = lax.rem(nch - 1, 2)
    lastt = lax.rem(nch - 1, 3)
    pltpu.make_async_copy(rows_v.at[lastb], a_sh.at[ei_v.at[lastt, 1]],
                          sems).wait()
    pltpu.make_async_copy(ones_v, c_sh.at[cidx_v.at[lastt]], sems).wait()

    plsc.subcore_barrier()

    for k in range(NWCHUNK):
        r0 = s * RPT + k * WCHUNK
        if k >= 2:
            pltpu.make_async_copy(rows_v.at[k % 2],
                                  a_out.at[c, pl.ds(0, WCHUNK)], sems).wait()
        pltpu.sync_copy(a_sh.at[pl.ds(r0, WCHUNK)], rows_v.at[k % 2])
        pltpu.async_copy(rows_v.at[k % 2], a_out.at[c, pl.ds(r0, WCHUNK)],
                         sems)
    pltpu.sync_copy(c_sh.at[pl.ds(s * ZSH, ZSH)], zcnt_v)
    pltpu.async_copy(zcnt_v, c_out.at[c, pl.ds(s * ZSH, ZSH)], sems)
    for k in range(NWCHUNK - 2, NWCHUNK):
        pltpu.make_async_copy(rows_v.at[k % 2],
                              a_out.at[c, pl.ds(0, WCHUNK)], sems).wait()
    pltpu.make_async_copy(zcnt_v, c_out.at[c, pl.ds(0, ZSH)], sems).wait()


def _combine_body(a_ref, c_ref, x_ref, w0_ref, w1_ref, w2_ref, wh_ref,
                  b0_ref, b1_ref, b2_ref, bh_ref, o_ref):
    a = a_ref[0] + a_ref[1]
    cnt = c_ref[0] + c_ref[1]
    ct = jnp.transpose(cnt, (1, 0))
    x = x_ref[...]
    corr = jnp.zeros_like(a)
    for t, (wr, br) in enumerate(((w0_ref, b0_ref), (w1_ref, b1_ref),
                                  (w2_ref, b2_ref))):
        yt = lax.dot_general(x, wr[...], (((1,), (1,)), ((), ())),
                             preferred_element_type=jnp.float32)
        corr = corr + ct[:, t:t + 1] * (yt - br[...])
    yh = lax.dot_general(x, wh_ref[...], (((1,), (1,)), ((), ())),
                         preferred_element_type=jnp.float32)
    o_ref[...] = a - corr + yh + bh_ref[...]


_sc_mesh = plsc.VectorSubcoreMesh(
    core_axis_name="c", subcore_axis_name="s", num_cores=NC, num_subcores=NS)

_edge_pass = functools.partial(
    pl.kernel,
    out_type=[jax.ShapeDtypeStruct((NC, AP, D), jnp.float32),
              jax.ShapeDtypeStruct((NC, CNT_LEN), jnp.float32)],
    mesh=_sc_mesh,
    scratch_types=[
        pltpu.VMEM((3, 2, CHUNK), jnp.int32),
        pltpu.VMEM((3, CHUNK), jnp.int32),
        pltpu.VMEM((2, CHUNK), jnp.int32),
        pltpu.VMEM((3, CHUNK), jnp.int32),
        pltpu.VMEM((CHUNK,), jnp.float32),
        pltpu.VMEM((2, CHUNK, D), jnp.float32),
        pltpu.VMEM((ZSH,), jnp.float32),
        pltpu.VMEM_SHARED((AP, D), jnp.float32),
        pltpu.VMEM_SHARED((CNT_LEN,), jnp.float32),
        pltpu.SemaphoreType.DMA,
        pltpu.SemaphoreType.DMA,
        pltpu.SemaphoreType.DMA,
    ],
)(_edge_body)


def kernel(n_in_feats, edge_index, e_feats, W0, b0, W1, b1, W2, b2, Wh, bh):
    x = n_in_feats
    wspec = pl.BlockSpec((D, D), lambda i: (0, 0))
    bspec = pl.BlockSpec((1, D), lambda i: (0, 0))

    y3 = pl.pallas_call(
        _ybuild_body,
        grid=(N // YBLK,),
        in_specs=[pl.BlockSpec((YBLK, D), lambda i: (i, 0)),
                  wspec, wspec, wspec],
        out_specs=pl.BlockSpec((3, YBLK, D), lambda i: (0, i, 0)),
        out_shape=jax.ShapeDtypeStruct((3, AP, D), jnp.float32),
    )(x, W0, W1, W2)

    a_part, cnt_flat = _edge_pass(y3.reshape(3 * AP, D), edge_index, e_feats)
    cnt3 = cnt_flat.reshape(NC, 3, AP)

    out = pl.pallas_call(
        _combine_body,
        grid=(AP // BLKN,),
        in_specs=[pl.BlockSpec((NC, BLKN, D), lambda i: (0, i, 0)),
                  pl.BlockSpec((NC, 3, BLKN), lambda i: (0, 0, i)),
                  pl.BlockSpec((BLKN, D), lambda i: (i, 0)),
                  wspec, wspec, wspec, wspec,
                  bspec, bspec, bspec, bspec],
        out_specs=pl.BlockSpec((BLKN, D), lambda i: (i, 0)),
        out_shape=jax.ShapeDtypeStruct((N, D), jnp.float32),
    )(a_part, cnt3, x, W0, W1, W2, Wh, b0.reshape(1, D), b1.reshape(1, D),
      b2.reshape(1, D), bh.reshape(1, D))
    return out


# speedup vs baseline: 1.0550x; 1.0274x over previous
import functools

import jax
import jax.numpy as jnp
from jax import lax
from jax.experimental import pallas as pl
from jax.experimental.pallas import tpu as pltpu
from jax.experimental.pallas import tpu_sc as plsc

N = 10000
E = 320000
D = 128

NC = 2
NS = 16
NW = NC * NS
CHUNK = 128
NCHT = E // CHUNK
NCH_BASE = NCHT // NW
NCH_REM = NCHT - NCH_BASE * NW

AP = 10240
RPT = AP // NS
WCHUNK = CHUNK
NWCHUNK = RPT // WCHUNK

CNT_LEN = 3 * AP
ZSH = CNT_LEN // NS

BLKN = 1024
YBLK = 2000


def _ybuild_body(x_ref, w0_ref, w1_ref, w2_ref, y_ref):
    x = x_ref[...]
    for t, wr in enumerate((w0_ref, w1_ref, w2_ref)):
        y_ref[t] = lax.dot_general(
            x, wr[...], (((1,), (1,)), ((), ())),
            preferred_element_type=jnp.float32)


def _edge_body(y_hbm, ei_hbm, ef_hbm, a_out, c_out,
               ei_v, ef_v, gidx_v, cidx_v, ones_v,
               rows_v, zcnt_v, a_sh, c_sh,
               semi, semg, sems):
    c = lax.axis_index("c")
    s = lax.axis_index("s")
    wid = c * NS + s
    nch = NCH_BASE + jnp.where(wid < NCH_REM, 1, 0)

    zf = jnp.zeros((16,), jnp.float32)
    for j in range(CHUNK // 16):
        ones_v[pl.ds(j * 16, 16)] = jnp.ones((16,), jnp.float32)

    def _zc(i, carry):
        zcnt_v[pl.ds(i * 16, 16)] = zf
        return carry
    lax.fori_loop(0, ZSH // 16, _zc, 0)
    pltpu.sync_copy(zcnt_v, c_sh.at[pl.ds(s * ZSH, ZSH)])

    def _calc_idx(buf, slot):
        for j in range(CHUNK // 16):
            sl = pl.ds(j * 16, 16)
            sv = ei_v[slot, 0, sl]
            dv = ei_v[slot, 1, sl]
            ev = ef_v[slot, sl]
            gidx_v[buf, sl] = ev * AP + sv
            cidx_v[slot, sl] = ev * AP + dv

    base0 = wid * CHUNK
    pltpu.sync_copy(ei_hbm.at[:, pl.ds(base0, CHUNK)], ei_v.at[0])
    pltpu.sync_copy(ef_hbm.at[pl.ds(base0, CHUNK)], ef_v.at[0])
    _calc_idx(0, 0)
    pltpu.async_copy(y_hbm.at[gidx_v.at[0]], rows_v.at[0], semg)
    base1 = (NW + wid) * CHUNK
    pltpu.async_copy(ei_hbm.at[:, pl.ds(base1, CHUNK)], ei_v.at[1], semi)
    pltpu.async_copy(ef_hbm.at[pl.ds(base1, CHUNK)], ef_v.at[1], semi)

    def _zr(r, carry):
        for j in range(D // 16):
            rows_v[1, r, pl.ds(j * 16, 16)] = zf
        return carry
    lax.fori_loop(0, CHUNK, _zr, 0)

    def _za(k, carry):
        pltpu.async_copy(rows_v.at[1], a_sh.at[pl.ds(s * RPT + k * CHUNK,
                                                     CHUNK)], sems)
        return carry
    lax.fori_loop(0, RPT // CHUNK, _za, 0)
    for k in range(RPT // CHUNK):
        pltpu.make_async_copy(rows_v.at[1], a_sh.at[pl.ds(0, CHUNK)],
                              sems).wait()

    plsc.subcore_barrier()

    def _chunk(g, carry):
        b = lax.rem(g, 2)
        nb = 1 - b
        t = lax.rem(g, 3)
        tn = lax.rem(g + 1, 3)
        tp = lax.rem(g + 2, 3)
        do_g1 = (g + 1) < nch
        do_l2 = (g + 2) < nch

        @pl.when(g > 0)
        def _drain_prev():
            pltpu.make_async_copy(rows_v.at[nb], a_sh.at[ei_v.at[tp, 1]],
                                  sems).wait()
            pltpu.make_async_copy(ones_v, c_sh.at[cidx_v.at[tp]], sems).wait()

        @pl.when(do_l2)
        def _pf_loads():
            base = ((g + 2) * NW + wid) * CHUNK
            pltpu.async_copy(ei_hbm.at[:, pl.ds(base, CHUNK)], ei_v.at[tp],
                             semi)
            pltpu.async_copy(ef_hbm.at[pl.ds(base, CHUNK)], ef_v.at[tp], semi)

        @pl.when(do_g1)
        def _pf_gather():
            pltpu.make_async_copy(ei_hbm.at[:, pl.ds(0, CHUNK)], ei_v.at[tn],
                                  semi).wait()
            pltpu.make_async_copy(ef_hbm.at[pl.ds(0, CHUNK)], ef_v.at[tn],
                                  semi).wait()
            _calc_idx(nb, tn)
            pltpu.async_copy(y_hbm.at[gidx_v.at[nb]], rows_v.at[nb], semg)

        pltpu.make_async_copy(y_hbm.at[pl.ds(0, CHUNK)], rows_v.at[b], semg).wait()
        pltpu.async_copy(rows_v.at[b], a_sh.at[ei_v.at[t, 1]], sems, add=True)
        pltpu.async_copy(ones_v, c_sh.at[cidx_v.at[t]], sems, add=True)
        return carry
    lax.fori_loop(0, nch, _chunk, 0)

    lastb = lax.rem(nch - 1, 2)
    lastt = lax.rem(nch - 1, 3)
    pltpu.make_async_copy(rows_v.at[lastb], a_sh.at[ei_v.at[lastt, 1]],
                          sems).wait()
    pltpu.make_async_copy(ones_v, c_sh.at[cidx_v.at[lastt]], sems).wait()

    plsc.subcore_barrier()

    for k in range(NWCHUNK):
        r0 = s * RPT + k * WCHUNK
        if k >= 2:
            pltpu.make_async_copy(rows_v.at[k % 2],
                                  a_out.at[c, pl.ds(0, WCHUNK)], sems).wait()
        pltpu.sync_copy(a_sh.at[pl.ds(r0, WCHUNK)], rows_v.at[k % 2])
        pltpu.async_copy(rows_v.at[k % 2], a_out.at[c, pl.ds(r0, WCHUNK)],
                         sems)
    pltpu.sync_copy(c_sh.at[pl.ds(s * ZSH, ZSH)], zcnt_v)
    pltpu.async_copy(zcnt_v, c_out.at[c, pl.ds(s * ZSH, ZSH)], sems)
    for k in range(NWCHUNK - 2, NWCHUNK):
        pltpu.make_async_copy(rows_v.at[k % 2],
                              a_out.at[c, pl.ds(0, WCHUNK)], sems).wait()
    pltpu.make_async_copy(zcnt_v, c_out.at[c, pl.ds(0, ZSH)], sems).wait()


def _combine_body(a_ref, c_ref, x_ref, w0_ref, w1_ref, w2_ref, wh_ref,
                  b0_ref, b1_ref, b2_ref, bh_ref, o_ref):
    a = a_ref[0] + a_ref[1]
    cnt = c_ref[0] + c_ref[1]
    ct = jnp.transpose(cnt, (1, 0))
    x = x_ref[...]
    corr = jnp.zeros_like(a)
    for t, (wr, br) in enumerate(((w0_ref, b0_ref), (w1_ref, b1_ref),
                                  (w2_ref, b2_ref))):
        yt = lax.dot_general(x, wr[...], (((1,), (1,)), ((), ())),
                             preferred_element_type=jnp.float32)
        corr = corr + ct[:, t:t + 1] * (yt - br[...])
    yh = lax.dot_general(x, wh_ref[...], (((1,), (1,)), ((), ())),
                         preferred_element_type=jnp.float32)
    o_ref[...] = a - corr + yh + bh_ref[...]


_sc_mesh = plsc.VectorSubcoreMesh(
    core_axis_name="c", subcore_axis_name="s", num_cores=NC, num_subcores=NS)

_edge_pass = functools.partial(
    pl.kernel,
    out_type=[jax.ShapeDtypeStruct((NC, AP, D), jnp.float32),
              jax.ShapeDtypeStruct((NC, CNT_LEN), jnp.float32)],
    mesh=_sc_mesh,
    scratch_types=[
        pltpu.VMEM((3, 2, CHUNK), jnp.int32),
        pltpu.VMEM((3, CHUNK), jnp.int32),
        pltpu.VMEM((2, CHUNK), jnp.int32),
        pltpu.VMEM((3, CHUNK), jnp.int32),
        pltpu.VMEM((CHUNK,), jnp.float32),
        pltpu.VMEM((2, CHUNK, D), jnp.float32),
        pltpu.VMEM((ZSH,), jnp.float32),
        pltpu.VMEM_SHARED((AP, D), jnp.float32),
        pltpu.VMEM_SHARED((CNT_LEN,), jnp.float32),
        pltpu.SemaphoreType.DMA,
        pltpu.SemaphoreType.DMA,
        pltpu.SemaphoreType.DMA,
    ],
)(_edge_body)


def kernel(n_in_feats, edge_index, e_feats, W0, b0, W1, b1, W2, b2, Wh, bh):
    x = n_in_feats
    wspec = pl.BlockSpec((D, D), lambda i: (0, 0))
    bspec = pl.BlockSpec((1, D), lambda i: (0, 0))

    y3 = pl.pallas_call(
        _ybuild_body,
        grid=(N // YBLK,),
        in_specs=[pl.BlockSpec((YBLK, D), lambda i: (i, 0)),
                  wspec, wspec, wspec],
        out_specs=pl.BlockSpec((3, YBLK, D), lambda i: (0, i, 0)),
        out_shape=jax.ShapeDtypeStruct((3, AP, D), jnp.float32),
    )(x, W0, W1, W2)

    a_part, cnt_flat = _edge_pass(y3.reshape(3 * AP, D), edge_index, e_feats)
    cnt3 = cnt_flat.reshape(NC, 3, AP)

    out = pl.pallas_call(
        _combine_body,
        grid=(AP // BLKN,),
        in_specs=[pl.BlockSpec((NC, BLKN, D), lambda i: (0, i, 0)),
                  pl.BlockSpec((NC, 3, BLKN), lambda i: (0, 0, i)),
                  pl.BlockSpec((BLKN, D), lambda i: (i, 0)),
                  wspec, wspec, wspec, wspec,
                  bspec, bspec, bspec, bspec],
        out_specs=pl.BlockSpec((BLKN, D), lambda i: (i, 0)),
        out_shape=jax.ShapeDtypeStruct((N, D), jnp.float32),
    )(a_part, cnt3, x, W0, W1, W2, Wh, b0.reshape(1, D), b1.reshape(1, D),
      b2.reshape(1, D), bh.reshape(1, D))
    return out


# speedup vs baseline: 1.0684x; 1.0127x over previous
import functools

import jax
import jax.numpy as jnp
from jax import lax
from jax.experimental import pallas as pl
from jax.experimental.pallas import tpu as pltpu
from jax.experimental.pallas import tpu_sc as plsc

N = 10000
E = 320000
D = 128

NC = 2
NS = 16
NW = NC * NS
CHUNK = 128
NCHT = E // CHUNK
NCH_BASE = NCHT // NW
NCH_REM = NCHT - NCH_BASE * NW

AP = 10240
RPT = AP // NS
WCHUNK = CHUNK
NWCHUNK = RPT // WCHUNK

CNT_LEN = 3 * AP
ZSH = CNT_LEN // NS

BLKN = 2048
YBLK = 2000


def _ybuild_body(x_ref, w0_ref, w1_ref, w2_ref, y_ref):
    x = x_ref[...]
    for t, wr in enumerate((w0_ref, w1_ref, w2_ref)):
        y_ref[t] = lax.dot_general(
            x, wr[...], (((1,), (1,)), ((), ())),
            preferred_element_type=jnp.float32)


def _edge_body(y_hbm, ei_hbm, ef_hbm, a_out, c_out,
               ei_v, ef_v, gidx_v, cidx_v, ones_v,
               rows_v, zcnt_v, a_sh, c_sh,
               semi, semg, sems):
    c = lax.axis_index("c")
    s = lax.axis_index("s")
    wid = c * NS + s
    nch = NCH_BASE + jnp.where(wid < NCH_REM, 1, 0)

    zf = jnp.zeros((16,), jnp.float32)
    for j in range(CHUNK // 16):
        ones_v[pl.ds(j * 16, 16)] = jnp.ones((16,), jnp.float32)

    def _zc(i, carry):
        zcnt_v[pl.ds(i * 16, 16)] = zf
        return carry
    lax.fori_loop(0, ZSH // 16, _zc, 0)
    pltpu.sync_copy(zcnt_v, c_sh.at[pl.ds(s * ZSH, ZSH)])

    def _calc_idx(buf, slot):
        for j in range(CHUNK // 16):
            sl = pl.ds(j * 16, 16)
            sv = ei_v[slot, 0, sl]
            dv = ei_v[slot, 1, sl]
            ev = ef_v[slot, sl]
            gidx_v[buf, sl] = ev * AP + sv
            cidx_v[slot, sl] = ev * AP + dv

    base0 = wid * CHUNK
    pltpu.sync_copy(ei_hbm.at[:, pl.ds(base0, CHUNK)], ei_v.at[0])
    pltpu.sync_copy(ef_hbm.at[pl.ds(base0, CHUNK)], ef_v.at[0])
    _calc_idx(0, 0)
    pltpu.async_copy(y_hbm.at[gidx_v.at[0]], rows_v.at[0], semg)
    base1 = (NW + wid) * CHUNK
    pltpu.async_copy(ei_hbm.at[:, pl.ds(base1, CHUNK)], ei_v.at[1], semi)
    pltpu.async_copy(ef_hbm.at[pl.ds(base1, CHUNK)], ef_v.at[1], semi)

    def _zr(r, carry):
        for j in range(D // 16):
            rows_v[1, r, pl.ds(j * 16, 16)] = zf
        return carry
    lax.fori_loop(0, CHUNK, _zr, 0)

    def _za(k, carry):
        pltpu.async_copy(rows_v.at[1], a_sh.at[pl.ds(s * RPT + k * CHUNK,
                                                     CHUNK)], sems)
        return carry
    lax.fori_loop(0, RPT // CHUNK, _za, 0)
    for k in range(RPT // CHUNK):
        pltpu.make_async_copy(rows_v.at[1], a_sh.at[pl.ds(0, CHUNK)],
                              sems).wait()

    plsc.subcore_barrier()

    def _chunk(g, carry):
        b = lax.rem(g, 2)
        nb = 1 - b
        t = lax.rem(g, 3)
        tn = lax.rem(g + 1, 3)
        tp = lax.rem(g + 2, 3)
        do_g1 = (g + 1) < nch
        do_l2 = (g + 2) < nch

        @pl.when(g > 0)
        def _drain_prev():
            pltpu.make_async_copy(rows_v.at[nb], a_sh.at[ei_v.at[tp, 1]],
                                  sems).wait()
            pltpu.make_async_copy(ones_v, c_sh.at[cidx_v.at[tp]], sems).wait()

        @pl.when(do_l2)
        def _pf_loads():
            base = ((g + 2) * NW + wid) * CHUNK
            pltpu.async_copy(ei_hbm.at[:, pl.ds(base, CHUNK)], ei_v.at[tp],
                             semi)
            pltpu.async_copy(ef_hbm.at[pl.ds(base, CHUNK)], ef_v.at[tp], semi)

        @pl.when(do_g1)
        def _pf_gather():
            pltpu.make_async_copy(ei_hbm.at[:, pl.ds(0, CHUNK)], ei_v.at[tn],
                                  semi).wait()
            pltpu.make_async_copy(ef_hbm.at[pl.ds(0, CHUNK)], ef_v.at[tn],
                                  semi).wait()
            _calc_idx(nb, tn)
            pltpu.async_copy(y_hbm.at[gidx_v.at[nb]], rows_v.at[nb], semg)

        pltpu.make_async_copy(y_hbm.at[pl.ds(0, CHUNK)], rows_v.at[b], semg).wait()
        pltpu.async_copy(rows_v.at[b], a_sh.at[ei_v.at[t, 1]], sems, add=True)
        pltpu.async_copy(ones_v, c_sh.at[cidx_v.at[t]], sems, add=True)
        return carry
    lax.fori_loop(0, nch, _chunk, 0)

    lastb = lax.rem(nch - 1, 2)
    lastt = lax.rem(nch - 1, 3)
    pltpu.make_async_copy(rows_v.at[lastb], a_sh.at[ei_v.at[lastt, 1]],
                          sems).wait()
    pltpu.make_async_copy(ones_v, c_sh.at[cidx_v.at[lastt]], sems).wait()

    plsc.subcore_barrier()

    for k in range(NWCHUNK):
        r0 = s * RPT + k * WCHUNK
        if k >= 2:
            pltpu.make_async_copy(rows_v.at[k % 2],
                                  a_out.at[c, pl.ds(0, WCHUNK)], sems).wait()
        pltpu.sync_copy(a_sh.at[pl.ds(r0, WCHUNK)], rows_v.at[k % 2])
        pltpu.async_copy(rows_v.at[k % 2], a_out.at[c, pl.ds(r0, WCHUNK)],
                         sems)
    pltpu.sync_copy(c_sh.at[pl.ds(s * ZSH, ZSH)], zcnt_v)
    pltpu.async_copy(zcnt_v, c_out.at[c, pl.ds(s * ZSH, ZSH)], sems)
    for k in range(NWCHUNK - 2, NWCHUNK):
        pltpu.make_async_copy(rows_v.at[k % 2],
                              a_out.at[c, pl.ds(0, WCHUNK)], sems).wait()
    pltpu.make_async_copy(zcnt_v, c_out.at[c, pl.ds(0, ZSH)], sems).wait()


def _combine_body(a_ref, c_ref, x_ref, w0_ref, w1_ref, w2_ref, wh_ref,
                  b0_ref, b1_ref, b2_ref, bh_ref, o_ref):
    a = a_ref[0] + a_ref[1]
    cnt = c_ref[0] + c_ref[1]
    ct = jnp.transpose(cnt, (1, 0))
    x = x_ref[...]
    corr = jnp.zeros_like(a)
    for t, (wr, br) in enumerate(((w0_ref, b0_ref), (w1_ref, b1_ref),
                                  (w2_ref, b2_ref))):
        yt = lax.dot_general(x, wr[...], (((1,), (1,)), ((), ())),
                             preferred_element_type=jnp.float32)
        corr = corr + ct[:, t:t + 1] * (yt - br[...])
    yh = lax.dot_general(x, wh_ref[...], (((1,), (1,)), ((), ())),
                         preferred_element_type=jnp.float32)
    o_ref[...] = a - corr + yh + bh_ref[...]


_sc_mesh = plsc.VectorSubcoreMesh(
    core_axis_name="c", subcore_axis_name="s", num_cores=NC, num_subcores=NS)

_edge_pass = functools.partial(
    pl.kernel,
    out_type=[jax.ShapeDtypeStruct((NC, AP, D), jnp.float32),
              jax.ShapeDtypeStruct((NC, CNT_LEN), jnp.float32)],
    mesh=_sc_mesh,
    scratch_types=[
        pltpu.VMEM((3, 2, CHUNK), jnp.int32),
        pltpu.VMEM((3, CHUNK), jnp.int32),
        pltpu.VMEM((2, CHUNK), jnp.int32),
        pltpu.VMEM((3, CHUNK), jnp.int32),
        pltpu.VMEM((CHUNK,), jnp.float32),
        pltpu.VMEM((2, CHUNK, D), jnp.float32),
        pltpu.VMEM((ZSH,), jnp.float32),
        pltpu.VMEM_SHARED((AP, D), jnp.float32),
        pltpu.VMEM_SHARED((CNT_LEN,), jnp.float32),
        pltpu.SemaphoreType.DMA,
        pltpu.SemaphoreType.DMA,
        pltpu.SemaphoreType.DMA,
    ],
)(_edge_body)


def kernel(n_in_feats, edge_index, e_feats, W0, b0, W1, b1, W2, b2, Wh, bh):
    x = n_in_feats
    wspec = pl.BlockSpec((D, D), lambda i: (0, 0))
    bspec = pl.BlockSpec((1, D), lambda i: (0, 0))

    y3 = pl.pallas_call(
        _ybuild_body,
        grid=(N // YBLK,),
        in_specs=[pl.BlockSpec((YBLK, D), lambda i: (i, 0)),
                  wspec, wspec, wspec],
        out_specs=pl.BlockSpec((3, YBLK, D), lambda i: (0, i, 0)),
        out_shape=jax.ShapeDtypeStruct((3, AP, D), jnp.float32),
    )(x, W0, W1, W2)

    a_part, cnt_flat = _edge_pass(y3.reshape(3 * AP, D), edge_index, e_feats)
    cnt3 = cnt_flat.reshape(NC, 3, AP)

    out = pl.pallas_call(
        _combine_body,
        grid=(AP // BLKN,),
        in_specs=[pl.BlockSpec((NC, BLKN, D), lambda i: (0, i, 0)),
                  pl.BlockSpec((NC, 3, BLKN), lambda i: (0, 0, i)),
                  pl.BlockSpec((BLKN, D), lambda i: (i, 0)),
                  wspec, wspec, wspec, wspec,
                  bspec, bspec, bspec, bspec],
        out_specs=pl.BlockSpec((BLKN, D), lambda i: (i, 0)),
        out_shape=jax.ShapeDtypeStruct((N, D), jnp.float32),
    )(a_part, cnt3, x, W0, W1, W2, Wh, b0.reshape(1, D), b1.reshape(1, D),
      b2.reshape(1, D), bh.reshape(1, D))
    return out
